# Initial kernel scaffold; baseline (speedup 1.0000x reference)
import jax
import jax.numpy as jnp
from jax.experimental import pallas as pl


def kernel(x, edge_index, W_l, b_l, W_r, W_out, b_out):
    raise NotImplementedError("write your pallas kernel here")


# speedup vs baseline: 6.1756x; 6.1756x over previous
import functools

import jax
import jax.numpy as jnp
from jax import lax
from jax.experimental import pallas as pl
from jax.experimental.pallas import tpu as pltpu
from jax.experimental.pallas import tpu_sc as plsc

_NC = 2
_NS = 16
_NW = _NC * _NS


@functools.lru_cache(maxsize=None)
def _make_agg(N, E, D, K):
    EW = E // _NW
    n_chunks = EW // K
    stripe = (N // _NS) // 8 * 8
    tail = N - stripe * _NS

    mesh = plsc.VectorSubcoreMesh(core_axis_name="c", subcore_axis_name="s")

    @functools.partial(
        pl.kernel,
        out_type=[
            jax.ShapeDtypeStruct((_NC * N, D), jnp.float32),
            jax.ShapeDtypeStruct((_NW * N,), jnp.float32),
        ],
        mesh=mesh,
        scratch_types=[
            pltpu.VMEM((K,), jnp.int32),
            pltpu.VMEM((K,), jnp.int32),
            pltpu.VMEM((K, D), jnp.float32),
            pltpu.VMEM((N,), jnp.float32),
            pltpu.VMEM_SHARED((N, D), jnp.float32),
            pltpu.SemaphoreType.DMA,
        ],
        compiler_params=pltpu.CompilerParams(needs_layout_passes=False),
    )
    def agg(src_h, dst_h, x_h, z_h, acc_out, deg_out,
            src_v, dst_v, rows_v, deg_v, acc_sh, sem):
        i32 = jnp.int32
        cid = lax.axis_index("c")
        sid = lax.axis_index("s")
        wid = sid * i32(_NC) + cid
        base = wid * i32(EW)

        soff = sid * i32(stripe)
        pltpu.sync_copy(z_h.at[pl.ds(soff, stripe)],
                        acc_sh.at[pl.ds(soff, stripe)])
        if tail:
            @pl.when(sid == _NS - 1)
            def _zero_tail():
                pltpu.sync_copy(z_h.at[pl.ds(_NS * stripe, tail)],
                                acc_sh.at[pl.ds(_NS * stripe, tail)])

        def zbody(i, carry):
            deg_v[pl.ds(i * i32(16), 16)] = jnp.zeros((16,), jnp.float32)
            return carry
        lax.fori_loop(i32(0), i32(N // 16), zbody, i32(0))

        plsc.subcore_barrier()

        ones = jnp.ones((16,), jnp.float32)

        def chunk(g, carry):
            off = pl.multiple_of(base + g * i32(K), 8)
            pltpu.sync_copy(src_h.at[pl.ds(off, K)], src_v)
            pltpu.sync_copy(dst_h.at[pl.ds(off, K)], dst_v)
            pltpu.async_copy(x_h.at[src_v], rows_v, sem).wait()
            pltpu.sync_copy(rows_v, acc_sh.at[dst_v], add=True)
            for j in range(K // 16):
                dvec = dst_v[pl.ds(j * 16, 16)]
                plsc.addupdate_scatter(deg_v, [dvec], ones)
            return carry
        lax.fori_loop(i32(0), i32(n_chunks), chunk, i32(0))

        plsc.subcore_barrier()

        pltpu.sync_copy(acc_sh.at[pl.ds(soff, stripe)],
                        acc_out.at[pl.ds(cid * i32(N) + soff, stripe)])
        if tail:
            @pl.when(sid == _NS - 1)
            def _out_tail():
                pltpu.sync_copy(
                    acc_sh.at[pl.ds(_NS * stripe, tail)],
                    acc_out.at[pl.ds(cid * i32(N) + _NS * stripe, tail)])
        pltpu.sync_copy(deg_v, deg_out.at[pl.ds(wid * i32(N), N)])

    return agg


def _dense_body(acc_ref, deg_ref, x_ref, wl_ref, bl_ref, wr_ref, wo_ref,
                bo_ref, out_ref):
    agg_sum = acc_ref[0] + acc_ref[1]
    deg = jnp.sum(deg_ref[...], axis=1, keepdims=True)
    agg = agg_sum * (1.0 / jnp.maximum(deg, 1.0))
    h = jnp.dot(agg, wl_ref[...], preferred_element_type=jnp.float32,
                precision=lax.Precision.HIGHEST)
    h = h + jnp.dot(x_ref[...], wr_ref[...], preferred_element_type=jnp.float32,
                    precision=lax.Precision.HIGHEST)
    h = h + bl_ref[...]
    h = jnp.maximum(h, 0.0)
    z = jnp.dot(h, wo_ref[...], preferred_element_type=jnp.float32,
                precision=lax.Precision.HIGHEST) + bo_ref[...]
    out_ref[...] = jax.nn.sigmoid(z)


@functools.lru_cache(maxsize=None)
def _make_dense(N, D, C, BN):
    grid = (N // BN,)

    def _z(i):
        return jnp.zeros_like(i)

    return pl.pallas_call(
        _dense_body,
        grid=grid,
        in_specs=[
            pl.BlockSpec((_NC, BN, D), lambda i: (_z(i), i, _z(i))),
            pl.BlockSpec((BN, _NW), lambda i: (i, _z(i))),
            pl.BlockSpec((BN, D), lambda i: (i, _z(i))),
            pl.BlockSpec((D, D), lambda i: (_z(i), _z(i))),
            pl.BlockSpec((1, D), lambda i: (_z(i), _z(i))),
            pl.BlockSpec((D, D), lambda i: (_z(i), _z(i))),
            pl.BlockSpec((D, C), lambda i: (_z(i), _z(i))),
            pl.BlockSpec((1, C), lambda i: (_z(i), _z(i))),
        ],
        out_specs=pl.BlockSpec((BN, C), lambda i: (i, _z(i))),
        out_shape=jax.ShapeDtypeStruct((N, C), jnp.float32),
    )


def kernel(x, edge_index, W_l, b_l, W_r, W_out, b_out):
    N, D = x.shape
    E = edge_index.shape[1]
    C = W_out.shape[0]
    ei = edge_index.astype(jnp.int32)
    src, dst = ei[0], ei[1]
    zeros = jnp.zeros((N, D), jnp.float32)

    acc_flat, deg_flat = _make_agg(N, E, D, 80)(src, dst, x, zeros)
    acc = acc_flat.reshape(_NC, N, D)
    deg = deg_flat.reshape(_NW, N).T

    out = _make_dense(N, D, C, 2000)(
        acc, deg, x,
        W_l.T, b_l.reshape(1, D), W_r.T, W_out.T, b_out.reshape(1, C))
    return out
